# baseline (speedup 1.0000x reference)
import jax
import jax.numpy as jnp
from jax import lax
from jax.experimental import pallas as pl
from jax.experimental.pallas import tpu as pltpu
from jax.experimental.pallas import tpu_sc as plsc

B = 64
D_IN = 8192
D_FEAT = 4096
D_OUT = 4096
K_MEM = 100
D_RETR = 7
BASE_LR = 2e-05

BF = 512
BO = 512
BI = 512

_F32 = jnp.float32


def _fwd1_body(x_ref, w1_ref, f_ref):
    f_ref[...] = jnp.dot(x_ref[...], w1_ref[...], preferred_element_type=_F32)


def _fwd2_body(f_ref, w2_ref, t_ref, o_ref, s_ref):
    j = pl.program_id(0)
    out = jnp.dot(f_ref[...], w2_ref[...], preferred_element_type=_F32)
    o_ref[...] = out
    t = t_ref[...]
    err = out - t

    @pl.when(j == 0)
    def _():
        s_ref[0, 0] = 0.0
        s_ref[0, 1] = 0.0

    s_ref[0, 0] += jnp.sum(err * err)
    s_ref[0, 1] += jnp.sum(t * t)


NSUB = 16
CW = D_FEAT // NSUB
NCV = CW // 16


def _sqrt16(x):
    xc = jnp.maximum(x, 1e-30)
    xi = lax.bitcast_convert_type(xc, jnp.int32)
    yi = jnp.int32(0x5F3759DF) - lax.shift_right_arithmetic(xi, 1)
    y = lax.bitcast_convert_type(yi, _F32)
    y = y * (1.5 - 0.5 * xc * y * y)
    y = y * (1.5 - 0.5 * xc * y * y)
    y = y * (1.5 - 0.5 * xc * y * y)
    return x * y


def _bc16(s):
    return jnp.full((16,), s, _F32)


def _retr_sc_body(f_hbm, mk_hbm, out_hbm, fbuf, mbuf, pbuf, allbuf, wstage,
                  wloc, pbuf2, sumbuf, outv, shared, shared2, shared3):
    s = lax.axis_index("s")
    cid = lax.axis_index("c")
    iota = lax.broadcasted_iota(jnp.int32, (16,), 0)
    zeros = jnp.zeros((16,), _F32)
    c0 = s * CW

    pltpu.sync_copy(f_hbm.at[:, pl.ds(c0, CW)], fbuf)
    pltpu.sync_copy(mk_hbm.at[pl.ds(B, K_MEM - B), pl.ds(c0, CW)], mbuf)

    def bodyf(i, acc):
        return tuple(acc[col] + fbuf[i, pl.ds(col * 16, 16)]
                     for col in range(NCV))

    ks = lax.fori_loop(0, B, bodyf, tuple(zeros for _ in range(NCV)))

    def bodym(i, acc):
        return tuple(acc[col] + mbuf[i, pl.ds(col * 16, 16)]
                     for col in range(NCV))

    ks = lax.fori_loop(0, K_MEM - B, bodym, ks)
    kv = [a * (1.0 / K_MEM) for a in ks]

    kp = zeros
    for col in range(NCV):
        kp = kp + kv[col] * kv[col]

    dvecs = []
    fvecs = []
    for g in range(4):
        def bodyg(l, carry, _g=g):
            dv, fv = carry
            i = _g * 16 + l
            dacc = zeros
            facc = zeros
            for col in range(NCV):
                row = fbuf[i, pl.ds(col * 16, 16)]
                dacc = dacc + row * kv[col]
                facc = facc + row * row
            lane = iota == l
            dv = jnp.where(lane, jnp.sum(dacc), dv)
            fv = jnp.where(lane, jnp.sum(facc), fv)
            return dv, fv

        dv, fv = lax.fori_loop(0, 16, bodyg, (zeros, zeros))
        dvecs.append(dv)
        fvecs.append(fv)

    for t in range(4):
        pbuf[pl.ds(16 * t, 16)] = dvecs[t]
        pbuf[pl.ds(64 + 16 * t, 16)] = fvecs[t]
    pbuf[pl.ds(128, 16)] = kp
    pltpu.sync_copy(pbuf, shared.at[s])
    plsc.subcore_barrier()

    @pl.when(s == 0)
    def _():
        pltpu.sync_copy(shared, allbuf)

        def bodyw(wi, carry):
            return tuple(carry[q] + allbuf[wi, pl.ds(16 * q, 16)]
                         for q in range(9))

        tot = lax.fori_loop(0, NSUB, bodyw, tuple(zeros for _ in range(9)))
        dots = tot[0:4]
        fn2 = tot[4:8]
        kn = _sqrt16(_bc16(jnp.sum(tot[8])))
        dvals = []
        for t in range(4):
            den = jnp.maximum(kn, 1e-8) * jnp.maximum(_sqrt16(fn2[t]), 1e-8)
            dvals.append(dots[t] / den)

        def bodyr(j, cnts):
            tj = j // 16
            lj = j - tj * 16
            dj_chunk = jnp.where(tj == 0, dvals[0],
                                 jnp.where(tj == 1, dvals[1],
                                           jnp.where(tj == 2, dvals[2],
                                                     dvals[3])))
            djs = _bc16(jnp.sum(jnp.where(iota == lj, dj_chunk, 0.0)))
            new = []
            for t in range(4):
                ivec = iota + 16 * t
                sel = (djs < dvals[t]) | ((djs == dvals[t]) & (j < ivec))
                new.append(cnts[t] + jnp.where(sel, 1.0, 0.0))
            return tuple(new)

        cnts = lax.fori_loop(0, B, bodyr, tuple(zeros for _ in range(4)))
        for t in range(4):
            w_t = jnp.where(cnts[t] < jnp.float32(D_RETR), 1.0, 0.0)
            rinv_t = 1.0 / jnp.maximum(_sqrt16(fn2[t]), 1e-12)
            wstage[pl.ds(16 * t, 16)] = w_t
            wstage[pl.ds(64 + 16 * t, 16)] = rinv_t
        pltpu.sync_copy(wstage, shared2)

    plsc.subcore_barrier()

    pltpu.sync_copy(shared2, wloc)
    wv = [wloc[pl.ds(16 * t, 16)] for t in range(4)]
    rv = [wloc[pl.ds(64 + 16 * t, 16)] for t in range(4)]

    def _lane_val(vlist, i):
        t = i // 16
        l = i - t * 16
        chunk = jnp.where(t == 0, vlist[0],
                          jnp.where(t == 1, vlist[1],
                                    jnp.where(t == 2, vlist[2], vlist[3])))
        return jnp.sum(jnp.where(iota == l, chunk, 0.0))

    def body_mk(i, sm):
        wi = _lane_val(wv, i)
        return tuple(sm[col] + wi * mbuf[i, pl.ds(col * 16, 16)]
                     for col in range(NCV))

    sm = lax.fori_loop(0, K_MEM - B, body_mk,
                       tuple(zeros for _ in range(NCV)))

    def body_f(i, sm):
        wi = _lane_val(wv, i + (K_MEM - B))
        return tuple(sm[col] + wi * fbuf[i, pl.ds(col * 16, 16)]
                     for col in range(NCV))

    sm = lax.fori_loop(0, 2 * B - K_MEM, body_f, sm)

    def body_fn(i, fn):
        ri = _lane_val(rv, i)
        return tuple(fn[col] + ri * fbuf[i, pl.ds(col * 16, 16)]
                     for col in range(NCV))

    fn = lax.fori_loop(0, B, body_fn, tuple(zeros for _ in range(NCV)))

    pv1 = zeros
    pv2 = zeros
    pv3 = zeros
    for col in range(NCV):
        smc = sm[col] * (1.0 / D_RETR)
        fnc = fn[col] * (1.0 / B)
        pv1 = pv1 + smc * smc
        pv2 = pv2 + fnc * fnc
        pv3 = pv3 + smc * fnc
    pbuf2[pl.ds(0, 16)] = pv1
    pbuf2[pl.ds(16, 16)] = pv2
    pbuf2[pl.ds(32, 16)] = pv3
    pltpu.sync_copy(pbuf2, shared3.at[s])
    plsc.subcore_barrier()

    @pl.when(s == 0)
    def _():
        pltpu.sync_copy(shared3, sumbuf)

        def bodyz(wi, carry):
            return tuple(carry[q] + sumbuf[wi, pl.ds(16 * q, 16)]
                         for q in range(3))

        tot = lax.fori_loop(0, NSUB, bodyz, (zeros, zeros, zeros))
        p1 = _bc16(jnp.sum(tot[0]))
        p2 = _bc16(jnp.sum(tot[1]))
        p3 = _bc16(jnp.sum(tot[2]))
        sqrt_p1 = _sqrt16(p1)
        denc = jnp.maximum(sqrt_p1, 1e-12)
        fdotc = p3 / denc
        cn = sqrt_p1 / denc
        fnn = _sqrt16(p2)
        cos = fdotc / (jnp.maximum(fnn, 1e-8) * jnp.maximum(cn, 1e-8))
        outv[...] = BASE_LR * jnp.exp((cos - 1.0) * 0.05)

        @pl.when(cid == 0)
        def _():
            pltpu.sync_copy(outv, out_hbm)


_retrieve_sc = pl.kernel(
    _retr_sc_body,
    out_type=jax.ShapeDtypeStruct((16,), _F32),
    mesh=plsc.VectorSubcoreMesh(core_axis_name="c", subcore_axis_name="s", num_cores=1),
    scratch_types=[
        pltpu.VMEM((B, CW), _F32),
        pltpu.VMEM((K_MEM - B, CW), _F32),
        pltpu.VMEM((144,), _F32),
        pltpu.VMEM((NSUB, 144), _F32),
        pltpu.VMEM((128,), _F32),
        pltpu.VMEM((128,), _F32),
        pltpu.VMEM((48,), _F32),
        pltpu.VMEM((NSUB, 48), _F32),
        pltpu.VMEM((16,), _F32),
        pltpu.VMEM_SHARED((NSUB, 144), _F32),
        pltpu.VMEM_SHARED((128,), _F32),
        pltpu.VMEM_SHARED((NSUB, 48), _F32),
    ],
    compiler_params=pltpu.CompilerParams(needs_layout_passes=False),
    cost_estimate=pl.CostEstimate(
        flops=4_000_000, bytes_accessed=50_000_000, transcendentals=1000),
)


def _bwd2_body(o_ref, t_ref, f_ref, w2_ref, s_ref, lr_ref, w2n_ref, gf_ref):
    n = jnp.float32(B * D_OUT)
    mse = s_ref[0, 0] / n
    mse_ref = jnp.maximum(s_ref[0, 1] / n, 1e-8)
    c = (0.5 / jnp.sqrt(mse) + 1.0 / mse_ref) * 2.0 / n
    lr = lr_ref[0, 0]
    g = c * (o_ref[...] - t_ref[...])
    gh = g.astype(jnp.bfloat16)
    fb = f_ref[...].astype(jnp.bfloat16)
    w2b = w2_ref[...]
    gw2 = lax.dot_general(fb, gh, (((0,), (0,)), ((), ())),
                          preferred_element_type=_F32)
    w2n_ref[...] = w2b - lr * gw2
    gf_ref[...] = lax.dot_general(gh, w2b.astype(jnp.bfloat16),
                                  (((1,), (1,)), ((), ())),
                                  preferred_element_type=_F32)


def _bwd1_body(x_ref, gf_ref, w1_ref, lr_ref, w1n_ref):
    lr = lr_ref[0, 0]
    xb = x_ref[...].astype(jnp.bfloat16)
    gf = gf_ref[...].astype(jnp.bfloat16)
    gw1 = lax.dot_general(xb, gf, (((0,), (0,)), ((), ())),
                          preferred_element_type=_F32)
    w1n_ref[...] = w1_ref[...] - lr * gw1


def kernel(inputs, target, mem_keys, W1, W2):
    feats = pl.pallas_call(
        _fwd1_body,
        grid=(D_FEAT // BF,),
        in_specs=[
            pl.BlockSpec((B, D_IN), lambda j: (0, 0)),
            pl.BlockSpec((D_IN, BF), lambda j: (0, j)),
        ],
        out_specs=pl.BlockSpec((B, BF), lambda j: (0, j)),
        out_shape=jax.ShapeDtypeStruct((B, D_FEAT), _F32),
        cost_estimate=pl.CostEstimate(
            flops=2 * B * D_IN * D_FEAT, bytes_accessed=137 * 2**20,
            transcendentals=0),
    )(inputs, W1)

    outputs, sums = pl.pallas_call(
        _fwd2_body,
        grid=(D_OUT // BO,),
        in_specs=[
            pl.BlockSpec((B, D_FEAT), lambda j: (0, 0)),
            pl.BlockSpec((D_FEAT, BO), lambda j: (0, j)),
            pl.BlockSpec((B, BO), lambda j: (0, j)),
        ],
        out_specs=[
            pl.BlockSpec((B, BO), lambda j: (0, j)),
            pl.BlockSpec((1, 2), lambda j: (0, 0), memory_space=pltpu.SMEM),
        ],
        out_shape=[
            jax.ShapeDtypeStruct((B, D_OUT), _F32),
            jax.ShapeDtypeStruct((1, 2), _F32),
        ],
        cost_estimate=pl.CostEstimate(
            flops=2 * B * D_FEAT * D_OUT, bytes_accessed=70 * 2**20,
            transcendentals=0),
    )(feats, W2, target)

    lr16 = _retrieve_sc(feats, mem_keys)
    lr = jnp.reshape(lr16[0:1], (1, 1))

    W2_new, gradfeats = pl.pallas_call(
        _bwd2_body,
        grid=(D_FEAT // BF,),
        in_specs=[
            pl.BlockSpec((B, D_OUT), lambda j: (0, 0)),
            pl.BlockSpec((B, D_OUT), lambda j: (0, 0)),
            pl.BlockSpec((B, BF), lambda j: (0, j)),
            pl.BlockSpec((BF, D_OUT), lambda j: (j, 0)),
            pl.BlockSpec((1, 2), lambda j: (0, 0), memory_space=pltpu.SMEM),
            pl.BlockSpec((1, 1), lambda j: (0, 0), memory_space=pltpu.SMEM),
        ],
        out_specs=[
            pl.BlockSpec((BF, D_OUT), lambda j: (j, 0)),
            pl.BlockSpec((B, BF), lambda j: (0, j)),
        ],
        out_shape=[
            jax.ShapeDtypeStruct((D_FEAT, D_OUT), _F32),
            jax.ShapeDtypeStruct((B, D_FEAT), _F32),
        ],
        cost_estimate=pl.CostEstimate(
            flops=4 * B * D_FEAT * D_OUT, bytes_accessed=132 * 2**20,
            transcendentals=0),
    )(outputs, target, feats, W2, sums, lr)

    W1_new = pl.pallas_call(
        _bwd1_body,
        grid=(D_IN // BI,),
        in_specs=[
            pl.BlockSpec((B, BI), lambda j: (0, j)),
            pl.BlockSpec((B, D_FEAT), lambda j: (0, 0)),
            pl.BlockSpec((BI, D_FEAT), lambda j: (j, 0)),
            pl.BlockSpec((1, 1), lambda j: (0, 0), memory_space=pltpu.SMEM),
        ],
        out_specs=pl.BlockSpec((BI, D_FEAT), lambda j: (j, 0)),
        out_shape=jax.ShapeDtypeStruct((D_IN, D_FEAT), _F32),
        cost_estimate=pl.CostEstimate(
            flops=2 * B * D_IN * D_FEAT, bytes_accessed=260 * 2**20,
            transcendentals=0),
    )(inputs, gradfeats, W1, lr)

    adjusted_lr = jnp.reshape(lr, ())
    return outputs, adjusted_lr, W1_new, W2_new


# speedup vs baseline: 1.0012x; 1.0012x over previous
import jax
import jax.numpy as jnp
from jax import lax
from jax.experimental import pallas as pl
from jax.experimental.pallas import tpu as pltpu
from jax.experimental.pallas import tpu_sc as plsc

B = 64
D_IN = 8192
D_FEAT = 4096
D_OUT = 4096
K_MEM = 100
D_RETR = 7
BASE_LR = 2e-05

BF = 512
BO = 512
BI = 512

_F32 = jnp.float32


def _fwd1_body(x_ref, w1_ref, f_ref):
    f_ref[...] = jnp.dot(x_ref[...], w1_ref[...], preferred_element_type=_F32)


def _fwd2_body(f_ref, w2_ref, t_ref, o_ref, s_ref):
    j = pl.program_id(0)
    out = jnp.dot(f_ref[...], w2_ref[...], preferred_element_type=_F32)
    o_ref[...] = out
    t = t_ref[...]
    err = out - t

    @pl.when(j == 0)
    def _():
        s_ref[0, 0] = 0.0
        s_ref[0, 1] = 0.0

    s_ref[0, 0] += jnp.sum(err * err)
    s_ref[0, 1] += jnp.sum(t * t)


NSUB = 16
CW = D_FEAT // NSUB
NCV = CW // 16


def _sqrt16(x):
    xc = jnp.maximum(x, 1e-30)
    xi = lax.bitcast_convert_type(xc, jnp.int32)
    yi = jnp.int32(0x5F3759DF) - lax.shift_right_arithmetic(xi, 1)
    y = lax.bitcast_convert_type(yi, _F32)
    y = y * (1.5 - 0.5 * xc * y * y)
    y = y * (1.5 - 0.5 * xc * y * y)
    y = y * (1.5 - 0.5 * xc * y * y)
    return x * y


def _bc16(s):
    return jnp.full((16,), s, _F32)


def _retr_sc_body(f_hbm, mk_hbm, out_hbm, fbuf, mbuf, pbuf, allbuf, wstage,
                  wloc, pbuf2, sumbuf, outv, shared, shared2, shared3):
    s = lax.axis_index("s")
    cid = lax.axis_index("c")
    iota = lax.broadcasted_iota(jnp.int32, (16,), 0)
    zeros = jnp.zeros((16,), _F32)
    c0 = s * CW

    pltpu.sync_copy(f_hbm.at[:, pl.ds(c0, CW)], fbuf)
    pltpu.sync_copy(mk_hbm.at[pl.ds(B, K_MEM - B), pl.ds(c0, CW)], mbuf)

    def bodyf(i, acc):
        return tuple(acc[col] + fbuf[i, pl.ds(col * 16, 16)]
                     for col in range(NCV))

    ks = lax.fori_loop(0, B, bodyf, tuple(zeros for _ in range(NCV)))

    def bodym(i, acc):
        return tuple(acc[col] + mbuf[i, pl.ds(col * 16, 16)]
                     for col in range(NCV))

    ks = lax.fori_loop(0, K_MEM - B, bodym, ks)
    kv = [a * (1.0 / K_MEM) for a in ks]

    kp = zeros
    for col in range(NCV):
        kp = kp + kv[col] * kv[col]

    dvecs = []
    fvecs = []
    for g in range(4):
        def bodyg(l, carry, _g=g):
            dv, fv = carry
            i = _g * 16 + l
            dacc = zeros
            facc = zeros
            for col in range(NCV):
                row = fbuf[i, pl.ds(col * 16, 16)]
                dacc = dacc + row * kv[col]
                facc = facc + row * row
            lane = iota == l
            dv = jnp.where(lane, jnp.sum(dacc), dv)
            fv = jnp.where(lane, jnp.sum(facc), fv)
            return dv, fv

        dv, fv = lax.fori_loop(0, 16, bodyg, (zeros, zeros))
        dvecs.append(dv)
        fvecs.append(fv)

    for t in range(4):
        pbuf[pl.ds(16 * t, 16)] = dvecs[t]
        pbuf[pl.ds(64 + 16 * t, 16)] = fvecs[t]
    pbuf[pl.ds(128, 16)] = kp
    pltpu.sync_copy(pbuf, shared.at[s])
    plsc.subcore_barrier()

    @pl.when(s == 0)
    def _():
        pltpu.sync_copy(shared, allbuf)

        def bodyw(wi, carry):
            return tuple(carry[q] + allbuf[wi, pl.ds(16 * q, 16)]
                         for q in range(9))

        tot = lax.fori_loop(0, NSUB, bodyw, tuple(zeros for _ in range(9)))
        dots = tot[0:4]
        fn2 = tot[4:8]
        kn = _sqrt16(_bc16(jnp.sum(tot[8])))
        dvals = []
        for t in range(4):
            den = jnp.maximum(kn, 1e-8) * jnp.maximum(_sqrt16(fn2[t]), 1e-8)
            dvals.append(dots[t] / den)

        def bodyr(j, cnts):
            tj = j // 16
            lj = j - tj * 16
            dj_chunk = jnp.where(tj == 0, dvals[0],
                                 jnp.where(tj == 1, dvals[1],
                                           jnp.where(tj == 2, dvals[2],
                                                     dvals[3])))
            djs = _bc16(jnp.sum(jnp.where(iota == lj, dj_chunk, 0.0)))
            new = []
            for t in range(4):
                ivec = iota + 16 * t
                sel = (djs < dvals[t]) | ((djs == dvals[t]) & (j < ivec))
                new.append(cnts[t] + jnp.where(sel, 1.0, 0.0))
            return tuple(new)

        cnts = lax.fori_loop(0, B, bodyr, tuple(zeros for _ in range(4)))
        for t in range(4):
            w_t = jnp.where(cnts[t] < jnp.float32(D_RETR), 1.0, 0.0)
            rinv_t = 1.0 / jnp.maximum(_sqrt16(fn2[t]), 1e-12)
            wstage[pl.ds(16 * t, 16)] = w_t
            wstage[pl.ds(64 + 16 * t, 16)] = rinv_t
        pltpu.sync_copy(wstage, shared2)

    plsc.subcore_barrier()

    pltpu.sync_copy(shared2, wloc)
    wv = [wloc[pl.ds(16 * t, 16)] for t in range(4)]
    rv = [wloc[pl.ds(64 + 16 * t, 16)] for t in range(4)]

    def _lane_val(vlist, i):
        t = i // 16
        l = i - t * 16
        chunk = jnp.where(t == 0, vlist[0],
                          jnp.where(t == 1, vlist[1],
                                    jnp.where(t == 2, vlist[2], vlist[3])))
        return jnp.sum(jnp.where(iota == l, chunk, 0.0))

    def body_mk(i, sm):
        wi = _lane_val(wv, i)
        return tuple(sm[col] + wi * mbuf[i, pl.ds(col * 16, 16)]
                     for col in range(NCV))

    sm = lax.fori_loop(0, K_MEM - B, body_mk,
                       tuple(zeros for _ in range(NCV)))

    def body_f(i, sm):
        wi = _lane_val(wv, i + (K_MEM - B))
        return tuple(sm[col] + wi * fbuf[i, pl.ds(col * 16, 16)]
                     for col in range(NCV))

    sm = lax.fori_loop(0, 2 * B - K_MEM, body_f, sm)

    def body_fn(i, fn):
        ri = _lane_val(rv, i)
        return tuple(fn[col] + ri * fbuf[i, pl.ds(col * 16, 16)]
                     for col in range(NCV))

    fn = lax.fori_loop(0, B, body_fn, tuple(zeros for _ in range(NCV)))

    pv1 = zeros
    pv2 = zeros
    pv3 = zeros
    for col in range(NCV):
        smc = sm[col] * (1.0 / D_RETR)
        fnc = fn[col] * (1.0 / B)
        pv1 = pv1 + smc * smc
        pv2 = pv2 + fnc * fnc
        pv3 = pv3 + smc * fnc
    pbuf2[pl.ds(0, 16)] = pv1
    pbuf2[pl.ds(16, 16)] = pv2
    pbuf2[pl.ds(32, 16)] = pv3
    pltpu.sync_copy(pbuf2, shared3.at[s])
    plsc.subcore_barrier()

    @pl.when(s == 0)
    def _():
        pltpu.sync_copy(shared3, sumbuf)

        def bodyz(wi, carry):
            return tuple(carry[q] + sumbuf[wi, pl.ds(16 * q, 16)]
                         for q in range(3))

        tot = lax.fori_loop(0, NSUB, bodyz, (zeros, zeros, zeros))
        p1 = _bc16(jnp.sum(tot[0]))
        p2 = _bc16(jnp.sum(tot[1]))
        p3 = _bc16(jnp.sum(tot[2]))
        sqrt_p1 = _sqrt16(p1)
        denc = jnp.maximum(sqrt_p1, 1e-12)
        fdotc = p3 / denc
        cn = sqrt_p1 / denc
        fnn = _sqrt16(p2)
        cos = fdotc / (jnp.maximum(fnn, 1e-8) * jnp.maximum(cn, 1e-8))
        outv[...] = BASE_LR * jnp.exp((cos - 1.0) * 0.05)

        @pl.when(cid == 0)
        def _():
            pltpu.sync_copy(outv, out_hbm)


_retrieve_sc = pl.kernel(
    _retr_sc_body,
    out_type=jax.ShapeDtypeStruct((16,), _F32),
    mesh=plsc.VectorSubcoreMesh(core_axis_name="c", subcore_axis_name="s", num_cores=1),
    scratch_types=[
        pltpu.VMEM((B, CW), _F32),
        pltpu.VMEM((K_MEM - B, CW), _F32),
        pltpu.VMEM((144,), _F32),
        pltpu.VMEM((NSUB, 144), _F32),
        pltpu.VMEM((128,), _F32),
        pltpu.VMEM((128,), _F32),
        pltpu.VMEM((48,), _F32),
        pltpu.VMEM((NSUB, 48), _F32),
        pltpu.VMEM((16,), _F32),
        pltpu.VMEM_SHARED((NSUB, 144), _F32),
        pltpu.VMEM_SHARED((128,), _F32),
        pltpu.VMEM_SHARED((NSUB, 48), _F32),
    ],
    compiler_params=pltpu.CompilerParams(needs_layout_passes=False),
    cost_estimate=pl.CostEstimate(
        flops=4_000_000, bytes_accessed=50_000_000, transcendentals=1000),
)


def _bwd2_body(o_ref, t_ref, f_ref, w2_ref, s_ref, lr_ref, w2n_ref, gf_ref):
    n = jnp.float32(B * D_OUT)
    mse = s_ref[0, 0] / n
    mse_ref = jnp.maximum(s_ref[0, 1] / n, 1e-8)
    c = (0.5 / jnp.sqrt(mse) + 1.0 / mse_ref) * 2.0 / n
    lr = lr_ref[0, 0]
    g = c * (o_ref[...] - t_ref[...])
    gh = g.astype(jnp.bfloat16)
    fb = f_ref[...].astype(jnp.bfloat16)
    w2b = w2_ref[...]
    gw2 = lax.dot_general(fb, gh, (((0,), (0,)), ((), ())),
                          preferred_element_type=_F32)
    w2n_ref[...] = w2b - lr * gw2
    gf_ref[...] = lax.dot_general(gh, w2b.astype(jnp.bfloat16),
                                  (((1,), (1,)), ((), ())),
                                  preferred_element_type=_F32)


def _bwd1_body(x_ref, gf_ref, w1_ref, lr_ref, w1n_ref):
    lr = lr_ref[0, 0]
    xb = x_ref[...].astype(jnp.bfloat16)
    gf = gf_ref[...].astype(jnp.bfloat16)
    gw1 = lax.dot_general(xb, gf, (((0,), (0,)), ((), ())),
                          preferred_element_type=_F32)
    w1n_ref[...] = w1_ref[...] - lr * gw1


def kernel(inputs, target, mem_keys, W1, W2):
    feats = pl.pallas_call(
        _fwd1_body,
        grid=(D_FEAT // BF,),
        in_specs=[
            pl.BlockSpec((B, D_IN), lambda j: (0, 0)),
            pl.BlockSpec((D_IN, BF), lambda j: (0, j)),
        ],
        out_specs=pl.BlockSpec((B, BF), lambda j: (0, j)),
        out_shape=jax.ShapeDtypeStruct((B, D_FEAT), _F32),
        cost_estimate=pl.CostEstimate(
            flops=2 * B * D_IN * D_FEAT, bytes_accessed=137 * 2**20,
            transcendentals=0),
    )(inputs, W1)

    lr16 = _retrieve_sc(feats, mem_keys)
    lr = jnp.reshape(lr16[0:1], (1, 1))

    outputs, sums = pl.pallas_call(
        _fwd2_body,
        grid=(D_OUT // BO,),
        in_specs=[
            pl.BlockSpec((B, D_FEAT), lambda j: (0, 0)),
            pl.BlockSpec((D_FEAT, BO), lambda j: (0, j)),
            pl.BlockSpec((B, BO), lambda j: (0, j)),
        ],
        out_specs=[
            pl.BlockSpec((B, BO), lambda j: (0, j)),
            pl.BlockSpec((1, 2), lambda j: (0, 0), memory_space=pltpu.SMEM),
        ],
        out_shape=[
            jax.ShapeDtypeStruct((B, D_OUT), _F32),
            jax.ShapeDtypeStruct((1, 2), _F32),
        ],
        cost_estimate=pl.CostEstimate(
            flops=2 * B * D_FEAT * D_OUT, bytes_accessed=70 * 2**20,
            transcendentals=0),
    )(feats, W2, target)


    W2_new, gradfeats = pl.pallas_call(
        _bwd2_body,
        grid=(D_FEAT // BF,),
        in_specs=[
            pl.BlockSpec((B, D_OUT), lambda j: (0, 0)),
            pl.BlockSpec((B, D_OUT), lambda j: (0, 0)),
            pl.BlockSpec((B, BF), lambda j: (0, j)),
            pl.BlockSpec((BF, D_OUT), lambda j: (j, 0)),
            pl.BlockSpec((1, 2), lambda j: (0, 0), memory_space=pltpu.SMEM),
            pl.BlockSpec((1, 1), lambda j: (0, 0), memory_space=pltpu.SMEM),
        ],
        out_specs=[
            pl.BlockSpec((BF, D_OUT), lambda j: (j, 0)),
            pl.BlockSpec((B, BF), lambda j: (0, j)),
        ],
        out_shape=[
            jax.ShapeDtypeStruct((D_FEAT, D_OUT), _F32),
            jax.ShapeDtypeStruct((B, D_FEAT), _F32),
        ],
        cost_estimate=pl.CostEstimate(
            flops=4 * B * D_FEAT * D_OUT, bytes_accessed=132 * 2**20,
            transcendentals=0),
    )(outputs, target, feats, W2, sums, lr)

    W1_new = pl.pallas_call(
        _bwd1_body,
        grid=(D_IN // BI,),
        in_specs=[
            pl.BlockSpec((B, BI), lambda j: (0, j)),
            pl.BlockSpec((B, D_FEAT), lambda j: (0, 0)),
            pl.BlockSpec((BI, D_FEAT), lambda j: (j, 0)),
            pl.BlockSpec((1, 1), lambda j: (0, 0), memory_space=pltpu.SMEM),
        ],
        out_specs=pl.BlockSpec((BI, D_FEAT), lambda j: (j, 0)),
        out_shape=jax.ShapeDtypeStruct((D_IN, D_FEAT), _F32),
        cost_estimate=pl.CostEstimate(
            flops=2 * B * D_IN * D_FEAT, bytes_accessed=260 * 2**20,
            transcendentals=0),
    )(inputs, gradfeats, W1, lr)

    adjusted_lr = jnp.reshape(lr, ())
    return outputs, adjusted_lr, W1_new, W2_new
